# baseline (device time: 179650 ns/iter reference)
import jax
import jax.numpy as jnp
from jax import lax
from jax.experimental import pallas as pl
from jax.experimental.pallas import tpu as pltpu

TILE_WIDTHS = (128, 256, 384, 384, 256, 128)
T = len(TILE_WIDTHS)
TILE_OFFS = tuple(sum(TILE_WIDTHS[:i]) for i in range(T))


def kernel(A, B):
    A = A.astype(jnp.bfloat16)
    m, k = A.shape
    n = B.shape[1]
    n_half = n // 2
    assert sum(TILE_WIDTHS) == n_half

    my_y_out = lax.axis_index("y")
    B_half = lax.dynamic_slice_in_dim(
        B, my_y_out * n_half, n_half, axis=1
    ).astype(jnp.bfloat16)

    def body(a_ref, b_ref, out_ref, recv_ref,
             send_sems_x, recv_sems_x, send_sems_y, recv_sems_y):
        my_x = lax.axis_index("x")
        my_y = lax.axis_index("y")
        x_nbr = (1 - my_x, my_y)
        y_nbr = (my_x, 1 - my_y)

        def out_tile(t):
            return pl.ds(my_y * n_half + TILE_OFFS[t], TILE_WIDTHS[t])

        def half_tile(t):
            return pl.ds(TILE_OFFS[t], TILE_WIDTHS[t])

        barrier_sem = pltpu.get_barrier_semaphore()
        for nbr in (x_nbr, y_nbr):
            pl.semaphore_signal(
                barrier_sem, inc=1, device_id=nbr,
                device_id_type=pl.DeviceIdType.MESH,
            )
        pl.semaphore_wait(barrier_sem, 2)

        rdma_x = []
        for t in range(T):
            out_ref[:, out_tile(t)] = jnp.dot(
                a_ref[...],
                b_ref[:, TILE_OFFS[t]:TILE_OFFS[t] + TILE_WIDTHS[t]],
                preferred_element_type=jnp.float32,
            ).astype(jnp.bfloat16)
            r = pltpu.make_async_remote_copy(
                src_ref=out_ref.at[:, out_tile(t)],
                dst_ref=recv_ref.at[:, half_tile(t)],
                send_sem=send_sems_x.at[t],
                recv_sem=recv_sems_x.at[t],
                device_id=x_nbr,
                device_id_type=pl.DeviceIdType.MESH,
            )
            r.start()
            rdma_x.append(r)

        rdma_y = []
        for t in range(T):
            rdma_x[t].wait()
            out_ref[:, out_tile(t)] = (
                out_ref[:, out_tile(t)] + recv_ref[:, half_tile(t)]
            )
            r = pltpu.make_async_remote_copy(
                src_ref=out_ref.at[:, out_tile(t)],
                dst_ref=out_ref.at[:, out_tile(t)],
                send_sem=send_sems_y.at[t],
                recv_sem=recv_sems_y.at[t],
                device_id=y_nbr,
                device_id_type=pl.DeviceIdType.MESH,
            )
            r.start()
            rdma_y.append(r)

        for t in range(T):
            rdma_y[t].wait()

    return pl.pallas_call(
        body,
        out_shape=jax.ShapeDtypeStruct((m, n), jnp.bfloat16),
        in_specs=[
            pl.BlockSpec(memory_space=pltpu.VMEM),
            pl.BlockSpec(memory_space=pltpu.VMEM),
        ],
        out_specs=pl.BlockSpec(memory_space=pltpu.VMEM),
        scratch_shapes=[
            pltpu.VMEM((m, n_half), jnp.bfloat16),
            pltpu.SemaphoreType.DMA((T,)),
            pltpu.SemaphoreType.DMA((T,)),
            pltpu.SemaphoreType.DMA((T,)),
            pltpu.SemaphoreType.DMA((T,)),
        ],
        compiler_params=pltpu.CompilerParams(
            collective_id=0, vmem_limit_bytes=100 * 1024 * 1024
        ),
    )(A, B_half)


# device time: 165966 ns/iter; 1.0825x vs baseline; 1.0825x over previous
import jax
import jax.numpy as jnp
from jax import lax
from jax.experimental import pallas as pl
from jax.experimental.pallas import tpu as pltpu

TILE_WIDTHS = (128, 256, 384, 384, 256, 128)
T = len(TILE_WIDTHS)
TILE_OFFS = tuple(sum(TILE_WIDTHS[:i]) for i in range(T))
MAX_W = max(TILE_WIDTHS)


def kernel(A, B):
    m, k = A.shape
    n = B.shape[1]
    n_half = n // 2
    assert sum(TILE_WIDTHS) == n_half
    k_half = k // 2

    def body(a_hbm, b_hbm, out_ref, a_stage, a_bf16, b_stage, b_bf16,
             recv_ref, a_sems, b_sems,
             send_sems_x, recv_sems_x, send_sems_y, recv_sems_y):
        my_x = lax.axis_index("x")
        my_y = lax.axis_index("y")
        x_nbr = (1 - my_x, my_y)
        y_nbr = (my_x, 1 - my_y)

        def out_tile(t):
            return pl.ds(my_y * n_half + TILE_OFFS[t], TILE_WIDTHS[t])

        def half_tile(t):
            return pl.ds(TILE_OFFS[t], TILE_WIDTHS[t])

        def b_hbm_tile(t):
            return pl.ds(my_y * n_half + TILE_OFFS[t], TILE_WIDTHS[t])

        def b_dma(t):
            return pltpu.make_async_copy(
                b_hbm.at[:, b_hbm_tile(t)],
                b_stage.at[t % 2, :, pl.ds(0, TILE_WIDTHS[t])],
                b_sems.at[t],
            )

        a_dma0 = pltpu.make_async_copy(
            a_hbm.at[:, pl.ds(0, k_half)], a_stage, a_sems.at[0]
        )
        a_dma0.start()
        b_dma(0).start()
        b_dma(1).start()

        barrier_sem = pltpu.get_barrier_semaphore()
        for nbr in (x_nbr, y_nbr):
            pl.semaphore_signal(
                barrier_sem, inc=1, device_id=nbr,
                device_id_type=pl.DeviceIdType.MESH,
            )

        a_dma0.wait()
        a_bf16[:, pl.ds(0, k_half)] = a_stage[...].astype(jnp.bfloat16)
        a_dma1 = pltpu.make_async_copy(
            a_hbm.at[:, pl.ds(k_half, k_half)], a_stage, a_sems.at[1]
        )
        a_dma1.start()
        a_dma1.wait()
        a_bf16[:, pl.ds(k_half, k_half)] = a_stage[...].astype(jnp.bfloat16)

        pl.semaphore_wait(barrier_sem, 2)

        rdma_x = []
        for t in range(T):
            b_dma(t).wait()
            w = TILE_WIDTHS[t]
            b_bf16[t % 2, :, pl.ds(0, w)] = (
                b_stage[t % 2, :, pl.ds(0, w)].astype(jnp.bfloat16)
            )
            if t + 2 < T:
                b_dma(t + 2).start()
            out_ref[:, out_tile(t)] = jnp.dot(
                a_bf16[...], b_bf16[t % 2, :, pl.ds(0, w)],
                preferred_element_type=jnp.float32,
            ).astype(jnp.bfloat16)
            r = pltpu.make_async_remote_copy(
                src_ref=out_ref.at[:, out_tile(t)],
                dst_ref=recv_ref.at[:, half_tile(t)],
                send_sem=send_sems_x.at[t],
                recv_sem=recv_sems_x.at[t],
                device_id=x_nbr,
                device_id_type=pl.DeviceIdType.MESH,
            )
            r.start()
            rdma_x.append(r)

        rdma_y = []
        for t in range(T):
            rdma_x[t].wait()
            out_ref[:, out_tile(t)] = (
                out_ref[:, out_tile(t)] + recv_ref[:, half_tile(t)]
            )
            r = pltpu.make_async_remote_copy(
                src_ref=out_ref.at[:, out_tile(t)],
                dst_ref=out_ref.at[:, out_tile(t)],
                send_sem=send_sems_y.at[t],
                recv_sem=recv_sems_y.at[t],
                device_id=y_nbr,
                device_id_type=pl.DeviceIdType.MESH,
            )
            r.start()
            rdma_y.append(r)

        for t in range(T):
            rdma_y[t].wait()

    return pl.pallas_call(
        body,
        out_shape=jax.ShapeDtypeStruct((m, n), jnp.bfloat16),
        in_specs=[
            pl.BlockSpec(memory_space=pl.ANY),
            pl.BlockSpec(memory_space=pl.ANY),
        ],
        out_specs=pl.BlockSpec(memory_space=pltpu.VMEM),
        scratch_shapes=[
            pltpu.VMEM((m, k // 2), jnp.float32),
            pltpu.VMEM((m, k), jnp.bfloat16),
            pltpu.VMEM((2, k, MAX_W), jnp.float32),
            pltpu.VMEM((2, k, MAX_W), jnp.bfloat16),
            pltpu.VMEM((m, n_half), jnp.bfloat16),
            pltpu.SemaphoreType.DMA((2,)),
            pltpu.SemaphoreType.DMA((T,)),
            pltpu.SemaphoreType.DMA((T,)),
            pltpu.SemaphoreType.DMA((T,)),
            pltpu.SemaphoreType.DMA((T,)),
            pltpu.SemaphoreType.DMA((T,)),
        ],
        compiler_params=pltpu.CompilerParams(
            collective_id=0, vmem_limit_bytes=100 * 1024 * 1024
        ),
    )(A, B)


# device time: 159946 ns/iter; 1.1232x vs baseline; 1.0376x over previous
import jax
import jax.numpy as jnp
from jax import lax
from jax.experimental import pallas as pl
from jax.experimental.pallas import tpu as pltpu

T = 6
W = 256


def kernel(A, B):
    m, k = A.shape
    n = B.shape[1]
    n_half = n // 2
    assert T * W == n_half
    k_half = k // 2

    def body(a_hbm, b_hbm, out_hbm, a_stage, a_bf16, b_stage, b_bf16,
             send_buf, recv_buf, recv_y_buf,
             a_sems, b_sems, out_sems, out_sems2,
             send_sems_x, recv_sems_x, send_sems_y, recv_sems_y):
        my_x = lax.axis_index("x")
        my_y = lax.axis_index("y")
        x_nbr = (1 - my_x, my_y)
        y_nbr = (my_x, 1 - my_y)

        def b_dma(t):
            return pltpu.make_async_copy(
                b_hbm.at[:, pl.ds(my_y * n_half + t * W, W)],
                b_stage.at[t % 2],
                b_sems.at[t],
            )

        a_dma0 = pltpu.make_async_copy(
            a_hbm.at[:, pl.ds(0, k_half)], a_stage, a_sems.at[0]
        )
        a_dma0.start()
        b_dma(0).start()
        b_dma(1).start()

        barrier_sem = pltpu.get_barrier_semaphore()
        for nbr in (x_nbr, y_nbr):
            pl.semaphore_signal(
                barrier_sem, inc=1, device_id=nbr,
                device_id_type=pl.DeviceIdType.MESH,
            )

        a_dma0.wait()
        a_bf16[:, pl.ds(0, k_half)] = a_stage[...].astype(jnp.bfloat16)
        a_dma1 = pltpu.make_async_copy(
            a_hbm.at[:, pl.ds(k_half, k_half)], a_stage, a_sems.at[1]
        )
        a_dma1.start()
        a_dma1.wait()
        a_bf16[:, pl.ds(k_half, k_half)] = a_stage[...].astype(jnp.bfloat16)

        pl.semaphore_wait(barrier_sem, 2)

        rdma_x = []
        for t in range(T):
            b_dma(t).wait()
            b_bf16[t % 2] = b_stage[t % 2].astype(jnp.bfloat16)
            if t + 2 < T:
                b_dma(t + 2).start()
            send_buf[t] = jnp.dot(
                a_bf16[...], b_bf16[t % 2],
                preferred_element_type=jnp.float32,
            ).astype(jnp.bfloat16)
            r = pltpu.make_async_remote_copy(
                src_ref=send_buf.at[t],
                dst_ref=recv_buf.at[t],
                send_sem=send_sems_x.at[t],
                recv_sem=recv_sems_x.at[t],
                device_id=x_nbr,
                device_id_type=pl.DeviceIdType.MESH,
            )
            r.start()
            rdma_x.append(r)

        rdma_y = []
        out_dmas = []
        for t in range(T):
            rdma_x[t].wait()
            send_buf[t] = send_buf[t] + recv_buf[t]
            r = pltpu.make_async_remote_copy(
                src_ref=send_buf.at[t],
                dst_ref=recv_y_buf.at[t],
                send_sem=send_sems_y.at[t],
                recv_sem=recv_sems_y.at[t],
                device_id=y_nbr,
                device_id_type=pl.DeviceIdType.MESH,
            )
            r.start()
            rdma_y.append(r)
            d = pltpu.make_async_copy(
                send_buf.at[t],
                out_hbm.at[:, pl.ds(my_y * n_half + t * W, W)],
                out_sems.at[t],
            )
            d.start()
            out_dmas.append(d)

        for t in range(T):
            rdma_y[t].wait_recv()
            d = pltpu.make_async_copy(
                recv_y_buf.at[t],
                out_hbm.at[:, pl.ds((1 - my_y) * n_half + t * W, W)],
                out_sems2.at[t],
            )
            d.start()
            out_dmas.append(d)

        for d in out_dmas:
            d.wait()
        for r in rdma_y:
            r.wait_send()

    return pl.pallas_call(
        body,
        out_shape=jax.ShapeDtypeStruct((m, n), jnp.bfloat16),
        in_specs=[
            pl.BlockSpec(memory_space=pl.ANY),
            pl.BlockSpec(memory_space=pl.ANY),
        ],
        out_specs=pl.BlockSpec(memory_space=pl.ANY),
        scratch_shapes=[
            pltpu.VMEM((m, k // 2), jnp.float32),
            pltpu.VMEM((m, k), jnp.bfloat16),
            pltpu.VMEM((2, k, W), jnp.float32),
            pltpu.VMEM((2, k, W), jnp.bfloat16),
            pltpu.VMEM((T, m, W), jnp.bfloat16),
            pltpu.VMEM((T, m, W), jnp.bfloat16),
            pltpu.VMEM((T, m, W), jnp.bfloat16),
            pltpu.SemaphoreType.DMA((2,)),
            pltpu.SemaphoreType.DMA((T,)),
            pltpu.SemaphoreType.DMA((T,)),
            pltpu.SemaphoreType.DMA((T,)),
            pltpu.SemaphoreType.DMA((T,)),
            pltpu.SemaphoreType.DMA((T,)),
            pltpu.SemaphoreType.DMA((T,)),
            pltpu.SemaphoreType.DMA((T,)),
        ],
        compiler_params=pltpu.CompilerParams(
            collective_id=0, vmem_limit_bytes=100 * 1024 * 1024
        ),
    )(A, B)
